# Initial kernel scaffold; baseline (speedup 1.0000x reference)
import jax
import jax.numpy as jnp
from jax.experimental import pallas as pl


def kernel(x, indices):
    raise NotImplementedError("write your pallas kernel here")


# speedup vs baseline: 89.4523x; 89.4523x over previous
import functools

import jax
import jax.numpy as jnp
from jax import lax
from jax.experimental import pallas as pl
from jax.experimental.pallas import tpu as pltpu
from jax.experimental.pallas import tpu_sc as plsc

_NUM_WORKERS = 32
_CH = 48
_LANES = 16


@functools.partial(jax.jit, static_argnums=(2, 3))
def _unpool(x2, idx2, hw, ow):
    planes = x2.shape[0]
    w = ow // 2
    h = hw // w
    tile_in = _CH * w
    tile_out = 2 * _CH * ow
    planes_per_worker = planes // _NUM_WORKERS
    chunks = h // _CH

    mesh = plsc.VectorSubcoreMesh(core_axis_name="c", subcore_axis_name="s")

    @functools.partial(
        pl.kernel,
        mesh=mesh,
        out_type=jax.ShapeDtypeStruct((planes, 4 * hw), jnp.float32),
        scratch_types=[
            pltpu.VMEM((tile_in,), jnp.float32),
            pltpu.VMEM((tile_in,), jnp.int32),
            pltpu.VMEM((tile_out,), jnp.float32),
        ],
        compiler_params=pltpu.CompilerParams(needs_layout_passes=False),
    )
    def body(x_hbm, idx_hbm, out_hbm, xv, iv, ov):
        wid = lax.axis_index("s") * 2 + lax.axis_index("c")
        zero16 = jnp.zeros((_LANES,), jnp.float32)

        def plane_loop(t, carry):
            p = wid * planes_per_worker + t

            def chunk_loop(ch, carry):
                i0 = ch * _CH
                pltpu.sync_copy(x_hbm.at[p, pl.ds(i0 * w, tile_in)], xv)
                pltpu.sync_copy(idx_hbm.at[p, pl.ds(i0 * w, tile_in)], iv)
                base = i0 * (2 * ow)

                def zero_loop(k, carry):
                    for u in range(8):
                        ov[pl.ds((k * 8 + u) * _LANES, _LANES)] = zero16
                    return carry

                lax.fori_loop(0, tile_out // _LANES // 8, zero_loop, 0)

                def scat_loop(k, carry):
                    for u in range(8):
                        kk = k * 8 + u
                        ids = iv[pl.ds(kk * _LANES, _LANES)] - base
                        vals = xv[pl.ds(kk * _LANES, _LANES)]
                        plsc.store_scatter(ov, [ids], vals)
                    return carry

                lax.fori_loop(0, tile_in // _LANES // 8, scat_loop, 0)

                pltpu.sync_copy(ov, out_hbm.at[p, pl.ds(base, tile_out)])
                return carry

            return lax.fori_loop(0, chunks, chunk_loop, carry)

        lax.fori_loop(0, planes_per_worker, plane_loop, 0)

    return body(x2, idx2)


def kernel(x, indices):
    n, c, h, w = x.shape
    x2 = x.reshape(n * c, h * w)
    idx2 = indices.reshape(n * c, h * w)
    out = _unpool(x2, idx2, h * w, 2 * w)
    return out.reshape(n, c, 2 * h, 2 * w)


# speedup vs baseline: 115.9718x; 1.2965x over previous
import functools

import jax
import jax.numpy as jnp
from jax import lax
from jax.experimental import pallas as pl
from jax.experimental.pallas import tpu as pltpu
from jax.experimental.pallas import tpu_sc as plsc

_NUM_WORKERS = 32
_CH = 48
_LANES = 16


@functools.partial(jax.jit, static_argnums=(2, 3))
def _unpool(x2, idx2, hw, ow):
    planes = x2.shape[0]
    w = ow // 2
    tile_in = _CH * w
    tile_out = 2 * _CH * ow
    planes_per_worker = planes // _NUM_WORKERS
    chunks = (hw // w) // _CH
    nt = planes_per_worker * chunks

    mesh = plsc.VectorSubcoreMesh(core_axis_name="c", subcore_axis_name="s")

    @functools.partial(
        pl.kernel,
        mesh=mesh,
        out_type=jax.ShapeDtypeStruct((planes, 4 * hw), jnp.float32),
        scratch_types=[
            pltpu.VMEM((tile_in,), jnp.float32),
            pltpu.VMEM((tile_in,), jnp.float32),
            pltpu.VMEM((tile_in,), jnp.int32),
            pltpu.VMEM((tile_in,), jnp.int32),
            pltpu.VMEM((tile_out,), jnp.float32),
            pltpu.VMEM((tile_out,), jnp.float32),
            pltpu.SemaphoreType.DMA,
            pltpu.SemaphoreType.DMA,
            pltpu.SemaphoreType.DMA,
            pltpu.SemaphoreType.DMA,
        ],
        compiler_params=pltpu.CompilerParams(needs_layout_passes=False),
    )
    def body(x_hbm, idx_hbm, out_hbm, xv0, xv1, iv0, iv1, ov0, ov1,
             si0, si1, so0, so1):
        xv = (xv0, xv1)
        iv = (iv0, iv1)
        ov = (ov0, ov1)
        si = (si0, si1)
        so = (so0, so1)
        wid = lax.axis_index("s") * 2 + lax.axis_index("c")
        zero16 = jnp.zeros((_LANES,), jnp.float32)

        def locate(t):
            p = wid * planes_per_worker + t // chunks
            i0 = (t % chunks) * _CH
            return p, i0

        def start_in(t, b):
            p, i0 = locate(t)
            pltpu.async_copy(x_hbm.at[p, pl.ds(i0 * w, tile_in)], xv[b], si[b])
            pltpu.async_copy(idx_hbm.at[p, pl.ds(i0 * w, tile_in)], iv[b], si[b])

        def wait_in(b):
            pltpu.make_async_copy(
                x_hbm.at[0, pl.ds(0, tile_in)], xv[b], si[b]).wait()
            pltpu.make_async_copy(
                idx_hbm.at[0, pl.ds(0, tile_in)], iv[b], si[b]).wait()

        def wait_out(b):
            pltpu.make_async_copy(
                ov[b], out_hbm.at[0, pl.ds(0, tile_out)], so[b]).wait()

        start_in(0, 0)
        start_in(1, 1)

        def group(g, carry):
            for b in range(2):
                t = g * 2 + b
                wait_in(b)

                @pl.when(g > 0)
                def _():
                    wait_out(b)

                def zero_loop(k, c):
                    for u in range(8):
                        ov[b][pl.ds((k * 8 + u) * _LANES, _LANES)] = zero16
                    return c

                lax.fori_loop(0, tile_out // _LANES // 8, zero_loop, 0)

                p, i0 = locate(t)
                base = i0 * 2 * ow

                def scat_loop(k, c):
                    for u in range(8):
                        kk = k * 8 + u
                        ids = iv[b][pl.ds(kk * _LANES, _LANES)] - base
                        vals = xv[b][pl.ds(kk * _LANES, _LANES)]
                        plsc.store_scatter(ov[b], [ids], vals)
                    return c

                lax.fori_loop(0, tile_in // _LANES // 8, scat_loop, 0)

                pltpu.async_copy(
                    ov[b], out_hbm.at[p, pl.ds(base, tile_out)], so[b])

                @pl.when(t + 2 < nt)
                def _():
                    start_in(t + 2, b)
            return carry

        lax.fori_loop(0, nt // 2, group, 0)
        wait_out(0)
        wait_out(1)

    return body(x2, idx2)


def kernel(x, indices):
    n, c, h, w = x.shape
    x2 = x.reshape(n * c, h * w)
    idx2 = indices.reshape(n * c, h * w)
    out = _unpool(x2, idx2, h * w, 2 * w)
    return out.reshape(n, c, 2 * h, 2 * w)


# speedup vs baseline: 172.7605x; 1.4897x over previous
import functools

import jax
import jax.numpy as jnp
from jax import lax
from jax.experimental import pallas as pl
from jax.experimental.pallas import tpu as pltpu
from jax.experimental.pallas import tpu_sc as plsc

_NUM_WORKERS = 32
_CH = 48
_LANES = 16


@jax.jit
def _unpool(x, idx):
    n, c, h, w = x.shape
    ow = 2 * w
    planes = n * c
    planes_per_worker = planes // _NUM_WORKERS
    chunks = h // _CH
    nt = planes_per_worker * chunks

    mesh = plsc.VectorSubcoreMesh(core_axis_name="c", subcore_axis_name="s")

    @functools.partial(
        pl.kernel,
        mesh=mesh,
        out_type=jax.ShapeDtypeStruct((n, c, 2 * h, ow), jnp.float32),
        scratch_types=[
            pltpu.VMEM((_CH, w), jnp.float32),
            pltpu.VMEM((_CH, w), jnp.float32),
            pltpu.VMEM((_CH, w), jnp.int32),
            pltpu.VMEM((_CH, w), jnp.int32),
            pltpu.VMEM((2 * _CH, ow), jnp.float32),
            pltpu.VMEM((2 * _CH, ow), jnp.float32),
            pltpu.SemaphoreType.DMA,
            pltpu.SemaphoreType.DMA,
            pltpu.SemaphoreType.DMA,
            pltpu.SemaphoreType.DMA,
        ],
        compiler_params=pltpu.CompilerParams(needs_layout_passes=False),
    )
    def body(x_hbm, idx_hbm, out_hbm, xv0, xv1, iv0, iv1, ov0, ov1,
             si0, si1, so0, so1):
        xv = (xv0, xv1)
        iv = (iv0, iv1)
        ov = (ov0, ov1)
        si = (si0, si1)
        so = (so0, so1)
        wid = lax.axis_index("s") * 2 + lax.axis_index("c")
        zero16 = jnp.zeros((_LANES,), jnp.float32)

        def locate(t):
            p = wid * planes_per_worker + t // chunks
            i0 = (t % chunks) * _CH
            return p // c, p % c, i0

        def start_in(t, b):
            nn, cc, i0 = locate(t)
            pltpu.async_copy(
                x_hbm.at[nn, cc, pl.ds(i0, _CH), :], xv[b], si[b])
            pltpu.async_copy(
                idx_hbm.at[nn, cc, pl.ds(i0, _CH), :], iv[b], si[b])

        def wait_in(b):
            pltpu.make_async_copy(
                x_hbm.at[0, 0, pl.ds(0, _CH), :], xv[b], si[b]).wait()
            pltpu.make_async_copy(
                idx_hbm.at[0, 0, pl.ds(0, _CH), :], iv[b], si[b]).wait()

        def wait_out(b):
            pltpu.make_async_copy(
                ov[b], out_hbm.at[0, 0, pl.ds(0, 2 * _CH), :], so[b]).wait()

        start_in(0, 0)
        start_in(1, 1)

        def group(g, carry):
            for b in range(2):
                t = g * 2 + b
                wait_in(b)

                @pl.when(g > 0)
                def _():
                    wait_out(b)

                def zero_loop(r2, c_):
                    for u in range(ow // _LANES):
                        ov[b][r2, pl.ds(u * _LANES, _LANES)] = zero16
                    return c_

                lax.fori_loop(0, 2 * _CH, zero_loop, 0)

                nn, cc, i0 = locate(t)

                def row_loop(r, c_):
                    b2 = (i0 + r) * 2 * ow
                    r2 = 2 * r
                    for u in range(w // _LANES):
                        ids = iv[b][r, pl.ds(u * _LANES, _LANES)]
                        rel = ids - b2
                        odd = (rel >= ow).astype(jnp.int32)
                        rows = r2 + odd
                        cols = rel - odd * ow
                        vals = xv[b][r, pl.ds(u * _LANES, _LANES)]
                        plsc.store_scatter(ov[b], [rows, cols], vals)
                    return c_

                lax.fori_loop(0, _CH, row_loop, 0)

                pltpu.async_copy(
                    ov[b], out_hbm.at[nn, cc, pl.ds(2 * i0, 2 * _CH), :],
                    so[b])

                @pl.when(t + 2 < nt)
                def _():
                    start_in(t + 2, b)
            return carry

        lax.fori_loop(0, nt // 2, group, 0)
        wait_out(0)
        wait_out(1)

    return body(x, idx)


def kernel(x, indices):
    return _unpool(x, indices)


# speedup vs baseline: 388.2621x; 2.2474x over previous
import functools

import jax
import jax.numpy as jnp
from jax import lax
from jax.experimental import pallas as pl
from jax.experimental.pallas import tpu as pltpu
from jax.experimental.pallas import tpu_sc as plsc

_NUM_WORKERS = 32
_CH = 48
_LANES = 16


@jax.jit
def _unpool(x, idx):
    n, c, h, w = x.shape
    ow = 2 * w
    planes = n * c
    planes_per_worker = planes // _NUM_WORKERS
    chunks = h // _CH
    nt = planes_per_worker * chunks

    mesh = plsc.VectorSubcoreMesh(core_axis_name="c", subcore_axis_name="s")

    @functools.partial(
        pl.kernel,
        mesh=mesh,
        out_type=jax.ShapeDtypeStruct((n, c, 2 * h, ow), jnp.float32),
        scratch_types=[
            pltpu.VMEM((_CH, w), jnp.float32),
            pltpu.VMEM((_CH, w), jnp.float32),
            pltpu.VMEM((_CH, w), jnp.int32),
            pltpu.VMEM((_CH, w), jnp.int32),
            pltpu.VMEM((2 * _CH, ow), jnp.float32),
            pltpu.VMEM((2 * _CH, ow), jnp.float32),
            pltpu.SemaphoreType.DMA,
            pltpu.SemaphoreType.DMA,
            pltpu.SemaphoreType.DMA,
            pltpu.SemaphoreType.DMA,
        ],
        compiler_params=pltpu.CompilerParams(needs_layout_passes=False),
    )
    def body(x_hbm, idx_hbm, out_hbm, xv0, xv1, iv0, iv1, ov0, ov1,
             si0, si1, so0, so1):
        xv = (xv0, xv1)
        iv = (iv0, iv1)
        ov = (ov0, ov1)
        si = (si0, si1)
        so = (so0, so1)
        wid = lax.axis_index("s") * 2 + lax.axis_index("c")
        zero16 = jnp.zeros((_LANES,), jnp.float32)

        def locate(t):
            p = wid * planes_per_worker + t // chunks
            i0 = (t % chunks) * _CH
            return p // c, p % c, i0

        def start_in(t, b):
            nn, cc, i0 = locate(t)
            pltpu.async_copy(
                x_hbm.at[nn, cc, pl.ds(i0, _CH), :], xv[b], si[b])
            pltpu.async_copy(
                idx_hbm.at[nn, cc, pl.ds(i0, _CH), :], iv[b], si[b])

        def wait_in(b):
            pltpu.make_async_copy(
                x_hbm.at[0, 0, pl.ds(0, _CH), :], xv[b], si[b]).wait()
            pltpu.make_async_copy(
                idx_hbm.at[0, 0, pl.ds(0, _CH), :], iv[b], si[b]).wait()

        def wait_out(b):
            pltpu.make_async_copy(
                ov[b], out_hbm.at[0, 0, pl.ds(0, 2 * _CH), :], so[b]).wait()

        start_in(0, 0)
        start_in(1, 1)

        def group(g, carry):
            for b in range(2):
                t = g * 2 + b
                wait_in(b)

                @pl.when(g > 0)
                def _():
                    wait_out(b)

                nn, cc, i0 = locate(t)

                def zero_loop(r2, c_):
                    for u in range(ow // _LANES):
                        ov[b][r2, pl.ds(u * _LANES, _LANES)] = zero16
                    return c_

                lax.fori_loop(0, 2 * _CH, zero_loop, 0)

                @plsc.parallel_loop(0, _CH, unroll=2)
                def row_loop(r):
                    r2 = 2 * r
                    b2 = (i0 + r) * 2 * ow
                    for u in range(w // _LANES):
                        ids = iv[b][r, pl.ds(u * _LANES, _LANES)]
                        rel = ids - b2
                        odd = rel >= ow
                        rows = jnp.where(odd, r2 + 1, r2)
                        cols = jnp.where(odd, rel - ow, rel)
                        vals = xv[b][r, pl.ds(u * _LANES, _LANES)]
                        plsc.store_scatter(ov[b], [rows, cols], vals)

                pltpu.async_copy(
                    ov[b], out_hbm.at[nn, cc, pl.ds(2 * i0, 2 * _CH), :],
                    so[b])

                @pl.when(t + 2 < nt)
                def _():
                    start_in(t + 2, b)
            return carry

        lax.fori_loop(0, nt // 2, group, 0)
        wait_out(0)
        wait_out(1)

    return body(x, idx)


def kernel(x, indices):
    return _unpool(x, indices)


# speedup vs baseline: 390.4790x; 1.0057x over previous
import functools

import jax
import jax.numpy as jnp
from jax import lax
from jax.experimental import pallas as pl
from jax.experimental.pallas import tpu as pltpu
from jax.experimental.pallas import tpu_sc as plsc

_NUM_WORKERS = 32
_CH = 48
_LANES = 16


@jax.jit
def _unpool(x, idx):
    n, c, h, w = x.shape
    ow = 2 * w
    planes = n * c
    planes_per_worker = planes // _NUM_WORKERS
    chunks = h // _CH
    nt = planes_per_worker * chunks

    mesh = plsc.VectorSubcoreMesh(core_axis_name="c", subcore_axis_name="s")

    @functools.partial(
        pl.kernel,
        mesh=mesh,
        out_type=jax.ShapeDtypeStruct((n, c, 2 * h, ow), jnp.float32),
        scratch_types=[
            pltpu.VMEM((_CH, w), jnp.float32),
            pltpu.VMEM((_CH, w), jnp.float32),
            pltpu.VMEM((_CH, w), jnp.int32),
            pltpu.VMEM((_CH, w), jnp.int32),
            pltpu.VMEM((2 * _CH, ow), jnp.float32),
            pltpu.VMEM((2 * _CH, ow), jnp.float32),
            pltpu.SemaphoreType.DMA,
            pltpu.SemaphoreType.DMA,
            pltpu.SemaphoreType.DMA,
            pltpu.SemaphoreType.DMA,
        ],
        compiler_params=pltpu.CompilerParams(needs_layout_passes=False),
    )
    def body(x_hbm, idx_hbm, out_hbm, xv0, xv1, iv0, iv1, ov0, ov1,
             si0, si1, so0, so1):
        xv = (xv0, xv1)
        iv = (iv0, iv1)
        ov = (ov0, ov1)
        si = (si0, si1)
        so = (so0, so1)
        wid = lax.axis_index("s") * 2 + lax.axis_index("c")
        zero16 = jnp.zeros((_LANES,), jnp.float32)

        def locate(t):
            p = wid * planes_per_worker + t // chunks
            i0 = (t % chunks) * _CH
            return p // c, p % c, i0

        def start_in(t, b):
            nn, cc, i0 = locate(t)
            pltpu.async_copy(
                x_hbm.at[nn, cc, pl.ds(i0, _CH), :], xv[b], si[b])
            pltpu.async_copy(
                idx_hbm.at[nn, cc, pl.ds(i0, _CH), :], iv[b], si[b])

        def wait_in(b):
            pltpu.make_async_copy(
                x_hbm.at[0, 0, pl.ds(0, _CH), :], xv[b], si[b]).wait()
            pltpu.make_async_copy(
                idx_hbm.at[0, 0, pl.ds(0, _CH), :], iv[b], si[b]).wait()

        def wait_out(b):
            pltpu.make_async_copy(
                ov[b], out_hbm.at[0, 0, pl.ds(0, 2 * _CH), :], so[b]).wait()

        start_in(0, 0)
        start_in(1, 1)

        def group(g, carry):
            for b in range(2):
                t = g * 2 + b
                wait_in(b)

                @pl.when(g > 0)
                def _():
                    wait_out(b)

                nn, cc, i0 = locate(t)

                @plsc.parallel_loop(0, 2 * _CH, unroll=2)
                def zero_loop(r2):
                    for u in range(ow // _LANES):
                        ov[b][r2, pl.ds(u * _LANES, _LANES)] = zero16

                @plsc.parallel_loop(0, _CH, unroll=2)
                def row_loop(r):
                    r2 = 2 * r
                    b2 = (i0 + r) * 2 * ow
                    for u in range(w // _LANES):
                        ids = iv[b][r, pl.ds(u * _LANES, _LANES)]
                        rel = ids - b2
                        odd = rel >= ow
                        rows = jnp.where(odd, r2 + 1, r2)
                        cols = jnp.where(odd, rel - ow, rel)
                        vals = xv[b][r, pl.ds(u * _LANES, _LANES)]
                        plsc.store_scatter(ov[b], [rows, cols], vals)

                pltpu.async_copy(
                    ov[b], out_hbm.at[nn, cc, pl.ds(2 * i0, 2 * _CH), :],
                    so[b])

                @pl.when(t + 2 < nt)
                def _():
                    start_in(t + 2, b)
            return carry

        lax.fori_loop(0, nt // 2, group, 0)
        wait_out(0)
        wait_out(1)

    return body(x, idx)


def kernel(x, indices):
    return _unpool(x, indices)
